# Initial kernel scaffold; baseline (speedup 1.0000x reference)
import jax
import jax.numpy as jnp
from jax.experimental import pallas as pl


def kernel(table, text, topic):
    raise NotImplementedError("write your pallas kernel here")


# speedup vs baseline: 3.3548x; 3.3548x over previous
import functools

import jax
import jax.numpy as jnp
from jax import lax
from jax.experimental import pallas as pl
from jax.experimental.pallas import tpu as pltpu
from jax.experimental.pallas import tpu_sc as plsc

VOCAB = 100000
D = 64
BATCH = 4096
TEXT_LEN = 200
TOPIC_LEN = 20

NC = 2
NS = 16
NW = NC * NS

CHUNK = 128

B1 = BATCH * TEXT_LEN
B2 = BATCH * TOPIC_LEN
PW1 = B1 // NW
PW2 = B2 // NW
NCH1 = PW1 // CHUNK
NCH2 = PW2 // CHUNK


def _make_kernel():
    mesh = plsc.VectorSubcoreMesh(core_axis_name="c", subcore_axis_name="s")

    @functools.partial(
        pl.kernel,
        mesh=mesh,
        compiler_params=pltpu.CompilerParams(use_tc_tiling_on_sc=False),
        out_type=(
            jax.ShapeDtypeStruct((B1, D), jnp.float32),
            jax.ShapeDtypeStruct((B2, D), jnp.float32),
        ),
        scratch_types=[
            pltpu.VMEM((NCH1, CHUNK), jnp.int32),
            pltpu.VMEM((NCH2, CHUNK), jnp.int32),
            pltpu.VMEM((2, CHUNK, D), jnp.float32),
            pltpu.SemaphoreType.DMA,
            pltpu.SemaphoreType.DMA,
        ],
    )
    def k(table, text, topic, out1, out2, idx1, idx2, rows, gsem, osem):
        wid = lax.axis_index("s") * NC + lax.axis_index("c")

        pltpu.sync_copy(text.at[wid], idx1)
        pltpu.sync_copy(topic.at[wid], idx2)

        def run(idx, nch, out, base):
            def body(j, _):
                buf = jax.lax.rem(j, 2)
                pltpu.async_copy(table.at[idx.at[j]], rows.at[buf], gsem).wait()
                pltpu.async_copy(
                    rows.at[buf], out.at[pl.ds(base + j * CHUNK, CHUNK)], osem
                ).wait()
                return 0

            lax.fori_loop(0, nch, body, 0)

        run(idx1, NCH1, out1, wid * PW1)
        run(idx2, NCH2, out2, wid * PW2)

    return k


_kern = _make_kernel()


def kernel(table, text, topic):
    text_r = text.reshape(NW, NCH1, CHUNK).astype(jnp.int32)
    topic_r = topic.reshape(NW, NCH2, CHUNK).astype(jnp.int32)
    out1, out2 = _kern(table, text_r, topic_r)
    return (
        out1.reshape(BATCH, TEXT_LEN, D),
        out2.reshape(BATCH, TOPIC_LEN, D),
    )


# speedup vs baseline: 4.0048x; 1.1937x over previous
import functools

import jax
import jax.numpy as jnp
from jax import lax
from jax.experimental import pallas as pl
from jax.experimental.pallas import tpu as pltpu
from jax.experimental.pallas import tpu_sc as plsc

VOCAB = 100000
D = 64
BATCH = 4096
TEXT_LEN = 200
TOPIC_LEN = 20

NC = 2
NS = 16
NW = NC * NS

CHUNK = 128
K1 = 4
K2 = 2

B1 = BATCH * TEXT_LEN
B2 = BATCH * TOPIC_LEN
PW1 = B1 // NW
PW2 = B2 // NW
NCH1 = PW1 // CHUNK
NCH2 = PW2 // CHUNK


def _make_kernel():
    mesh = plsc.VectorSubcoreMesh(core_axis_name="c", subcore_axis_name="s")

    @functools.partial(
        pl.kernel,
        mesh=mesh,
        compiler_params=pltpu.CompilerParams(use_tc_tiling_on_sc=False),
        out_type=(
            jax.ShapeDtypeStruct((B1, D), jnp.float32),
            jax.ShapeDtypeStruct((B2, D), jnp.float32),
        ),
        scratch_types=[
            pltpu.VMEM((NCH1, CHUNK), jnp.int32),
            pltpu.VMEM((NCH2, CHUNK), jnp.int32),
            pltpu.VMEM((2, K1 * CHUNK, D), jnp.float32),
            [pltpu.SemaphoreType.DMA] * 2,
            [pltpu.SemaphoreType.DMA] * 2,
        ],
    )
    def k(table, text, topic, out1, out2, idx1, idx2, rows, gsem, osem):
        wid = lax.axis_index("s") * NC + lax.axis_index("c")

        pltpu.sync_copy(text.at[wid], idx1)
        pltpu.sync_copy(topic.at[wid], idx2)

        def run(idx, nch, out, base, kk):
            ngroups = nch // kk

            def fire_g(t, s):
                for i in range(kk):
                    pltpu.async_copy(
                        table.at[idx.at[t * kk + i]],
                        rows.at[s, pl.ds(i * CHUNK, CHUNK)],
                        gsem[s],
                    )

            def drain_g(t, s):
                for i in range(kk):
                    pltpu.make_async_copy(
                        table.at[idx.at[t * kk + i]],
                        rows.at[s, pl.ds(i * CHUNK, CHUNK)],
                        gsem[s],
                    ).wait()

            def wb(t, s):
                return pltpu.make_async_copy(
                    rows.at[s, pl.ds(0, kk * CHUNK)],
                    out.at[pl.ds(base + t * kk * CHUNK, kk * CHUNK)],
                    osem[s],
                )

            fire_g(0, 0)
            fire_g(1, 1)
            drain_g(0, 0)
            wb(0, 0).start()

            def body(p, _):
                t0 = 2 + 2 * p
                drain_g(t0 - 1, 1)
                wb(t0 - 1, 1).start()
                wb(t0 - 2, 0).wait()
                fire_g(t0, 0)
                drain_g(t0, 0)
                wb(t0, 0).start()
                wb(t0 - 1, 1).wait()
                fire_g(t0 + 1, 1)
                return 0

            lax.fori_loop(0, (ngroups - 2) // 2, body, 0)

            drain_g(ngroups - 1, 1)
            wb(ngroups - 1, 1).start()
            wb(ngroups - 2, 0).wait()
            wb(ngroups - 1, 1).wait()

        run(idx1, NCH1, out1, wid * PW1, K1)
        run(idx2, NCH2, out2, wid * PW2, K2)

    return k


_kern = _make_kernel()


def kernel(table, text, topic):
    text_r = text.reshape(NW, NCH1, CHUNK).astype(jnp.int32)
    topic_r = topic.reshape(NW, NCH2, CHUNK).astype(jnp.int32)
    out1, out2 = _kern(table, text_r, topic_r)
    return (
        out1.reshape(BATCH, TEXT_LEN, D),
        out2.reshape(BATCH, TOPIC_LEN, D),
    )
